# baseline (speedup 1.0000x reference)
import jax
import jax.numpy as jnp
from jax.experimental import pallas as pl
from jax.experimental.pallas import tpu as pltpu

_E = 8
_H = 2048
_I = 4096
_T = 8192
_EPS = 1e-6
_ALPHA = 1.702
_LIMIT = 7.0

_BT = 512
_BN2 = 256
_NK = _I // _BN2
_NB = _T // _BT
_NT = _NB + _E - 1


def _ffn_kernel(g_ref, m_ref, s_ref, e_ref,
                x_ref, wn_ref, wu_ref, wd_ref, out_ref,
                xbf_ref, acc_ref):
    t = pl.program_id(0)
    k = pl.program_id(1)

    @pl.when(k == 0)
    def _norm():
        xf = x_ref[...]
        ms = jnp.mean(xf * xf, axis=-1, keepdims=True)
        xn = xf * jax.lax.rsqrt(ms + _EPS) * (wn_ref[0] + 1.0)
        xbf_ref[...] = xn.astype(jnp.bfloat16)

    xb = xbf_ref[...]
    wu = wu_ref[0]
    wg = wu[:, :_H]
    wl = wu[:, _H:]
    dn = (((1,), (1,)), ((), ()))
    u_glu = jax.lax.dot_general(xb, wg, dn, preferred_element_type=jnp.float32)
    u_lin = jax.lax.dot_general(xb, wl, dn, preferred_element_type=jnp.float32)
    u_glu = jnp.minimum(u_glu, _LIMIT)
    u_lin = jnp.clip(u_lin, -_LIMIT, _LIMIT)
    act = u_glu * jax.nn.sigmoid(_ALPHA * u_glu) * (u_lin + 1.0)
    part = jax.lax.dot_general(act.astype(jnp.bfloat16), wd_ref[0], dn,
                               preferred_element_type=jnp.float32)

    @pl.when(k == 0)
    def _init():
        acc_ref[...] = part

    @pl.when(k > 0)
    def _acc():
        acc_ref[...] += part

    @pl.when(k == _NK - 1)
    def _flush():
        rows = m_ref[t] * _BT + jax.lax.broadcasted_iota(jnp.int32, (_BT, 1), 0)
        mask = (rows >= s_ref[t]) & (rows < e_ref[t])
        contrib = jnp.where(mask, acc_ref[...], 0.0)
        m_prev = m_ref[jnp.maximum(t - 1, 0)]
        first = (t == 0) | (m_ref[t] != m_prev)

        @pl.when(first)
        def _():
            out_ref[...] = contrib

        @pl.when(jnp.logical_not(first))
        def _():
            out_ref[...] += contrib


def _route(mapping):
    m32 = mapping.astype(jnp.int32)
    off = jnp.searchsorted(
        m32, jnp.arange(_E + 1, dtype=jnp.int32), side="left").astype(jnp.int32)
    sizes = off[1:] - off[:-1]
    tf = off[:-1] // _BT
    tl = (off[1:] - 1) // _BT
    cnt = jnp.where(sizes > 0, tl - tf + 1, 0).astype(jnp.int32)
    cum = jnp.concatenate(
        [jnp.zeros((1,), jnp.int32), jnp.cumsum(cnt, dtype=jnp.int32)])
    total = cum[-1]
    i = jnp.arange(_NT, dtype=jnp.int32)
    ii = jnp.minimum(i, total - 1)
    g = (jnp.searchsorted(cum, ii, side="right").astype(jnp.int32) - 1)
    m = tf[g] + (ii - cum[g])
    pad = i >= total
    row_s = jnp.where(pad, 0, jnp.maximum(off[g], m * _BT))
    row_e = jnp.where(pad, 0, jnp.minimum(off[g + 1], (m + 1) * _BT))
    return g, m, row_s, row_e


def kernel(x, modality_mapping, w_norm, W_up, W_down):
    g, m, row_s, row_e = _route(modality_mapping)
    wn2 = w_norm.reshape(_E, 1, _H)
    wu3 = W_up.reshape(_E, _I, 2 * _H).astype(jnp.bfloat16)
    wdb = W_down.astype(jnp.bfloat16)

    grid_spec = pltpu.PrefetchScalarGridSpec(
        num_scalar_prefetch=4,
        grid=(_NT, _NK),
        in_specs=[
            pl.BlockSpec((_BT, _H), lambda t, k, g, m, s, e: (m[t], 0)),
            pl.BlockSpec((1, 1, _H), lambda t, k, g, m, s, e: (g[t], 0, 0)),
            pl.BlockSpec((1, _BN2, 2 * _H), lambda t, k, g, m, s, e: (g[t], k, 0)),
            pl.BlockSpec((1, _H, _BN2), lambda t, k, g, m, s, e: (g[t], 0, k)),
        ],
        out_specs=pl.BlockSpec((_BT, _H), lambda t, k, g, m, s, e: (m[t], 0)),
        scratch_shapes=[
            pltpu.VMEM((_BT, _H), jnp.bfloat16),
            pltpu.VMEM((_BT, _H), jnp.float32),
        ],
    )
    return pl.pallas_call(
        _ffn_kernel,
        grid_spec=grid_spec,
        out_shape=jax.ShapeDtypeStruct((_T, _H), jnp.float32),
        compiler_params=pltpu.CompilerParams(
            dimension_semantics=("arbitrary", "arbitrary"),
        ),
    )(g, m, row_s, row_e, x, wn2, wu3, wdb)


# speedup vs baseline: 1.0847x; 1.0847x over previous
import jax
import jax.numpy as jnp
from jax.experimental import pallas as pl
from jax.experimental.pallas import tpu as pltpu

_E = 8
_H = 2048
_I = 4096
_T = 8192
_EPS = 1e-6
_ALPHA = 1.702
_LIMIT = 7.0

_BT = 512
_BN2 = 256
_NK = _I // _BN2
_NB = _T // _BT
_NT = _NB + _E - 1


def _ffn_kernel(g_ref, m_ref, s_ref, e_ref,
                x_ref, wn_ref, wu_ref, wd_ref, out_ref,
                xbf_ref, acc_ref, ring_ref):
    t = pl.program_id(0)
    k = pl.program_id(1)

    @pl.when(k == 0)
    def _norm():
        xf = x_ref[...]
        ms = jnp.mean(xf * xf, axis=-1, keepdims=True)
        xn = xf * jax.lax.rsqrt(ms + _EPS) * (wn_ref[0] + 1.0)
        xbf_ref[...] = xn.astype(jnp.bfloat16)

    dn = (((1,), (1,)), ((), ()))

    part = jax.lax.dot_general(ring_ref[(k + 1) % 2], wd_ref[0].astype(jnp.bfloat16),
                               dn, preferred_element_type=jnp.float32)

    xb = xbf_ref[...]
    wu = wu_ref[0].astype(jnp.bfloat16)
    u_glu = jax.lax.dot_general(xb, wu[:, :_H], dn,
                                preferred_element_type=jnp.float32)
    u_lin = jax.lax.dot_general(xb, wu[:, _H:], dn,
                                preferred_element_type=jnp.float32)
    u_glu = jnp.minimum(u_glu, _LIMIT)
    u_lin = jnp.clip(u_lin, -_LIMIT, _LIMIT)
    act = u_glu * jax.nn.sigmoid(_ALPHA * u_glu) * (u_lin + 1.0)
    ring_ref[k % 2] = act.astype(jnp.bfloat16)

    @pl.when(k == 1)
    def _init():
        acc_ref[...] = part

    @pl.when(k > 1)
    def _acc():
        acc_ref[...] += part

    @pl.when(k == _NK)
    def _flush():
        rows = m_ref[t] * _BT + jax.lax.broadcasted_iota(jnp.int32, (_BT, 1), 0)
        mask = (rows >= s_ref[t]) & (rows < e_ref[t])
        contrib = jnp.where(mask, acc_ref[...], 0.0)
        m_prev = m_ref[jnp.maximum(t - 1, 0)]
        first = (t == 0) | (m_ref[t] != m_prev)

        @pl.when(first)
        def _():
            out_ref[...] = contrib

        @pl.when(jnp.logical_not(first))
        def _():
            out_ref[...] += contrib


def _route(mapping):
    m32 = mapping.astype(jnp.int32)
    off = jnp.searchsorted(
        m32, jnp.arange(_E + 1, dtype=jnp.int32), side="left").astype(jnp.int32)
    sizes = off[1:] - off[:-1]
    tf = off[:-1] // _BT
    tl = (off[1:] - 1) // _BT
    cnt = jnp.where(sizes > 0, tl - tf + 1, 0).astype(jnp.int32)
    cum = jnp.concatenate(
        [jnp.zeros((1,), jnp.int32), jnp.cumsum(cnt, dtype=jnp.int32)])
    total = cum[-1]
    i = jnp.arange(_NT, dtype=jnp.int32)
    ii = jnp.minimum(i, total - 1)
    g = (jnp.searchsorted(cum, ii, side="right").astype(jnp.int32) - 1)
    m = tf[g] + (ii - cum[g])
    pad = i >= total
    row_s = jnp.where(pad, 0, jnp.maximum(off[g], m * _BT))
    row_e = jnp.where(pad, 0, jnp.minimum(off[g + 1], (m + 1) * _BT))
    return g, m, row_s, row_e


def kernel(x, modality_mapping, w_norm, W_up, W_down):
    g, m, row_s, row_e = _route(modality_mapping)
    wn2 = w_norm.reshape(_E, 1, _H)
    wu3 = W_up.reshape(_E, _I, 2 * _H)

    grid_spec = pltpu.PrefetchScalarGridSpec(
        num_scalar_prefetch=4,
        grid=(_NT, _NK + 1),
        in_specs=[
            pl.BlockSpec((_BT, _H), lambda t, k, g, m, s, e: (m[t], 0)),
            pl.BlockSpec((1, 1, _H), lambda t, k, g, m, s, e: (g[t], 0, 0)),
            pl.BlockSpec((1, _BN2, 2 * _H),
                         lambda t, k, g, m, s, e: (g[t], jnp.minimum(k, _NK - 1), 0)),
            pl.BlockSpec((1, _H, _BN2),
                         lambda t, k, g, m, s, e: (g[t], 0, jnp.maximum(k - 1, 0))),
        ],
        out_specs=pl.BlockSpec((_BT, _H), lambda t, k, g, m, s, e: (m[t], 0)),
        scratch_shapes=[
            pltpu.VMEM((_BT, _H), jnp.bfloat16),
            pltpu.VMEM((_BT, _H), jnp.float32),
            pltpu.VMEM((2, _BT, _BN2), jnp.bfloat16),
        ],
    )
    return pl.pallas_call(
        _ffn_kernel,
        grid_spec=grid_spec,
        out_shape=jax.ShapeDtypeStruct((_T, _H), jnp.float32),
        compiler_params=pltpu.CompilerParams(
            dimension_semantics=("arbitrary", "arbitrary"),
        ),
    )(g, m, row_s, row_e, x, wn2, wu3, W_down)


# speedup vs baseline: 1.1781x; 1.0861x over previous
import jax
import jax.numpy as jnp
from jax.experimental import pallas as pl
from jax.experimental.pallas import tpu as pltpu

_E = 8
_H = 2048
_I = 4096
_T = 8192
_EPS = 1e-6
_ALPHA = 1.702
_LIMIT = 7.0

_BT = 512
_BN2 = 512
_NK = _I // _BN2
_NB = _T // _BT
_NT = _NB + _E - 1


def _ffn_kernel(g_ref, m_ref, s_ref, e_ref,
                x_ref, wn_ref, wu_ref, wd_ref, out_ref,
                xbf_ref, acc_ref, ring_ref):
    t = pl.program_id(0)
    k = pl.program_id(1)

    @pl.when(k == 0)
    def _norm():
        xf = x_ref[...]
        ms = jnp.mean(xf * xf, axis=-1, keepdims=True)
        xn = xf * jax.lax.rsqrt(ms + _EPS) * (wn_ref[0] + 1.0)
        xbf_ref[...] = xn.astype(jnp.bfloat16)

    dn = (((1,), (1,)), ((), ()))

    part = jax.lax.dot_general(ring_ref[(k + 1) % 2], wd_ref[0].astype(jnp.bfloat16),
                               dn, preferred_element_type=jnp.float32)

    xb = xbf_ref[...]
    wu = wu_ref[0].astype(jnp.bfloat16)
    u_glu = jax.lax.dot_general(xb, wu[:, :_H], dn,
                                preferred_element_type=jnp.float32)
    u_lin = jax.lax.dot_general(xb, wu[:, _H:], dn,
                                preferred_element_type=jnp.float32)
    u_glu = jnp.minimum(u_glu, _LIMIT)
    u_lin = jnp.clip(u_lin, -_LIMIT, _LIMIT)
    act = u_glu * jax.nn.sigmoid(_ALPHA * u_glu) * (u_lin + 1.0)
    ring_ref[k % 2] = act.astype(jnp.bfloat16)

    @pl.when(k == 1)
    def _init():
        acc_ref[...] = part

    @pl.when(k > 1)
    def _acc():
        acc_ref[...] += part

    @pl.when(k == _NK)
    def _flush():
        rows = m_ref[t] * _BT + jax.lax.broadcasted_iota(jnp.int32, (_BT, 1), 0)
        mask = (rows >= s_ref[t]) & (rows < e_ref[t])
        contrib = jnp.where(mask, acc_ref[...], 0.0)
        m_prev = m_ref[jnp.maximum(t - 1, 0)]
        first = (t == 0) | (m_ref[t] != m_prev)

        @pl.when(first)
        def _():
            out_ref[...] = contrib

        @pl.when(jnp.logical_not(first))
        def _():
            out_ref[...] += contrib


def _route(mapping):
    m32 = mapping.astype(jnp.int32)
    off = jnp.searchsorted(
        m32, jnp.arange(_E + 1, dtype=jnp.int32), side="left").astype(jnp.int32)
    sizes = off[1:] - off[:-1]
    tf = off[:-1] // _BT
    tl = (off[1:] - 1) // _BT
    cnt = jnp.where(sizes > 0, tl - tf + 1, 0).astype(jnp.int32)
    cum = jnp.concatenate(
        [jnp.zeros((1,), jnp.int32), jnp.cumsum(cnt, dtype=jnp.int32)])
    total = cum[-1]
    i = jnp.arange(_NT, dtype=jnp.int32)
    ii = jnp.minimum(i, total - 1)
    g = (jnp.searchsorted(cum, ii, side="right").astype(jnp.int32) - 1)
    m = tf[g] + (ii - cum[g])
    pad = i >= total
    row_s = jnp.where(pad, 0, jnp.maximum(off[g], m * _BT))
    row_e = jnp.where(pad, 0, jnp.minimum(off[g + 1], (m + 1) * _BT))
    return g, m, row_s, row_e


def kernel(x, modality_mapping, w_norm, W_up, W_down):
    g, m, row_s, row_e = _route(modality_mapping)
    wn2 = w_norm.reshape(_E, 1, _H)
    wu3 = W_up.reshape(_E, _I, 2 * _H)

    grid_spec = pltpu.PrefetchScalarGridSpec(
        num_scalar_prefetch=4,
        grid=(_NT, _NK + 1),
        in_specs=[
            pl.BlockSpec((_BT, _H), lambda t, k, g, m, s, e: (m[t], 0)),
            pl.BlockSpec((1, 1, _H), lambda t, k, g, m, s, e: (g[t], 0, 0)),
            pl.BlockSpec((1, _BN2, 2 * _H),
                         lambda t, k, g, m, s, e: (g[t], jnp.minimum(k, _NK - 1), 0)),
            pl.BlockSpec((1, _H, _BN2),
                         lambda t, k, g, m, s, e: (g[t], 0, jnp.maximum(k - 1, 0))),
        ],
        out_specs=pl.BlockSpec((_BT, _H), lambda t, k, g, m, s, e: (m[t], 0)),
        scratch_shapes=[
            pltpu.VMEM((_BT, _H), jnp.bfloat16),
            pltpu.VMEM((_BT, _H), jnp.float32),
            pltpu.VMEM((2, _BT, _BN2), jnp.bfloat16),
        ],
    )
    return pl.pallas_call(
        _ffn_kernel,
        grid_spec=grid_spec,
        out_shape=jax.ShapeDtypeStruct((_T, _H), jnp.float32),
        compiler_params=pltpu.CompilerParams(
            dimension_semantics=("arbitrary", "arbitrary"),
        ),
    )(g, m, row_s, row_e, x, wn2, wu3, W_down)
